# Initial kernel scaffold; baseline (speedup 1.0000x reference)
import jax
import jax.numpy as jnp
from jax.experimental import pallas as pl


def kernel(x, edge_index, W1_l, b1_l, W1_r, W2_l, b2_l, W2_r):
    raise NotImplementedError("write your pallas kernel here")


# speedup vs baseline: 5.2002x; 5.2002x over previous
import jax
import jax.numpy as jnp
from jax import lax
from jax.experimental import pallas as pl
from jax.experimental.pallas import tpu as pltpu
from jax.experimental.pallas import tpu_sc as plsc

NC = 2
NS = 16
NW = NC * NS
LANES = 16
CHUNK = 80


def _make_agg(N, Npad, D, E):
    ept = E // NW
    nchunk = ept // CHUNK
    rpt = Npad // NS
    mesh = plsc.VectorSubcoreMesh(core_axis_name="c", subcore_axis_name="s")

    def body(y_hbm, src_hbm, dst_hbm, zeros_hbm, out_hbm,
             idx_v, didx_v, rows_v, acc, sem):
        cid = lax.axis_index("c")
        sid = lax.axis_index("s")
        wid = sid * NC + cid
        stripe = pl.ds(sid * rpt, rpt)

        pltpu.sync_copy(zeros_hbm.at[stripe], acc.at[stripe])
        plsc.subcore_barrier()

        base = wid * ept

        def chunk_body(c, carry):
            off = pl.multiple_of(base + c * CHUNK, 8)
            pltpu.sync_copy(src_hbm.at[pl.ds(off, CHUNK)], idx_v)
            pltpu.sync_copy(dst_hbm.at[pl.ds(off, CHUNK)], didx_v)
            pltpu.async_copy(y_hbm.at[idx_v], rows_v, sem).wait()
            pltpu.sync_copy(rows_v, acc.at[didx_v], add=True)
            return carry

        lax.fori_loop(0, nchunk, chunk_body, 0)
        plsc.subcore_barrier()

        pltpu.sync_copy(acc.at[stripe], out_hbm.at[cid, stripe])

    return pl.kernel(
        body,
        out_type=jax.ShapeDtypeStruct((NC, Npad, D), jnp.float32),
        mesh=mesh,
        scratch_types=[pltpu.VMEM((CHUNK,), jnp.int32),
                       pltpu.VMEM((CHUNK,), jnp.int32),
                       pltpu.VMEM((CHUNK, D), jnp.float32),
                       pltpu.VMEM_SHARED((Npad, D), jnp.float32),
                       pltpu.SemaphoreType.DMA])


def _make_cnt(N, E):
    ept = E // NW
    mesh = plsc.VectorSubcoreMesh(core_axis_name="c", subcore_axis_name="s")

    def body(dst_hbm, out_hbm, didx_v, cnt_v):
        cid = lax.axis_index("c")
        sid = lax.axis_index("s")
        wid = sid * NC + cid
        zv = jnp.zeros((LANES,), jnp.float32)

        def zbody(i, carry):
            cnt_v[pl.ds(i * LANES, LANES)] = zv
            return carry

        lax.fori_loop(0, N // LANES, zbody, 0)

        base = wid * ept
        ones = jnp.ones((LANES,), jnp.float32)

        def chunk_body(c, carry):
            off = pl.multiple_of(base + c * CHUNK, 8)
            pltpu.sync_copy(dst_hbm.at[pl.ds(off, CHUNK)], didx_v)
            for i in range(CHUNK // LANES):
                iv = didx_v[pl.ds(i * LANES, LANES)]
                plsc.addupdate_scatter(cnt_v, [iv], ones)
            return carry

        lax.fori_loop(0, ept // CHUNK, chunk_body, 0)
        pltpu.sync_copy(cnt_v, out_hbm.at[wid])

    return pl.kernel(
        body,
        out_type=jax.ShapeDtypeStruct((NW, N), jnp.float32),
        mesh=mesh,
        scratch_types=[pltpu.VMEM((CHUNK,), jnp.int32),
                       pltpu.VMEM((N,), jnp.float32)],
        compiler_params=pltpu.CompilerParams(needs_layout_passes=False))


_DOT = (((1,), (1,)), ((), ()))


def _tc_pre(x, W_l, W_r, b):
    N, D = x.shape
    BR = 1000

    def body(x_ref, wl_ref, wr_ref, b_ref, y_ref, r_ref):
        xb = x_ref[...]
        y_ref[...] = lax.dot_general(xb, wl_ref[...], _DOT,
                                     preferred_element_type=jnp.float32)
        r_ref[...] = lax.dot_general(xb, wr_ref[...], _DOT,
                                     preferred_element_type=jnp.float32) + b_ref[...]

    return pl.pallas_call(
        body,
        grid=(N // BR,),
        in_specs=[pl.BlockSpec((BR, D), lambda i: (i, 0)),
                  pl.BlockSpec((D, D), lambda i: (0, 0)),
                  pl.BlockSpec((D, D), lambda i: (0, 0)),
                  pl.BlockSpec((1, D), lambda i: (0, 0))],
        out_specs=[pl.BlockSpec((BR, D), lambda i: (i, 0)),
                   pl.BlockSpec((BR, D), lambda i: (i, 0))],
        out_shape=[jax.ShapeDtypeStruct((N, D), jnp.float32)] * 2,
    )(x, W_l, W_r, b.reshape(1, D))


def _tc_mid(s1, cntT, r1, W2_l, W2_r, b2):
    N, D = r1.shape
    BR = 1000

    def body(s_ref, c_ref, r_ref, wl_ref, wr_ref, b_ref, y_ref, rr_ref):
        s = s_ref[0] + s_ref[1]
        c = jnp.sum(c_ref[...], axis=1, keepdims=True)
        pre = s / jnp.maximum(c, 1.0) + r_ref[...]
        nrm = jnp.sqrt(jnp.sum(pre * pre, axis=1, keepdims=True))
        h = jnp.maximum(pre / jnp.maximum(nrm, 1e-12), 0.0)
        y_ref[...] = lax.dot_general(h, wl_ref[...], _DOT,
                                     preferred_element_type=jnp.float32)
        rr_ref[...] = lax.dot_general(h, wr_ref[...], _DOT,
                                      preferred_element_type=jnp.float32) + b_ref[...]

    return pl.pallas_call(
        body,
        grid=(N // BR,),
        in_specs=[pl.BlockSpec((NC, BR, D), lambda i: (0, i, 0)),
                  pl.BlockSpec((BR, NW), lambda i: (i, 0)),
                  pl.BlockSpec((BR, D), lambda i: (i, 0)),
                  pl.BlockSpec((D, D), lambda i: (0, 0)),
                  pl.BlockSpec((D, D), lambda i: (0, 0)),
                  pl.BlockSpec((1, D), lambda i: (0, 0))],
        out_specs=[pl.BlockSpec((BR, D), lambda i: (i, 0)),
                   pl.BlockSpec((BR, D), lambda i: (i, 0))],
        out_shape=[jax.ShapeDtypeStruct((N, D), jnp.float32)] * 2,
    )(s1, cntT, r1, W2_l, W2_r, b2.reshape(1, D))


def _tc_post(s2, cntT, r2):
    N, D = r2.shape
    BR = 1000

    def body(s_ref, c_ref, r_ref, o_ref):
        s = s_ref[0] + s_ref[1]
        c = jnp.sum(c_ref[...], axis=1, keepdims=True)
        o_ref[...] = s / jnp.maximum(c, 1.0) + r_ref[...]

    return pl.pallas_call(
        body,
        grid=(N // BR,),
        in_specs=[pl.BlockSpec((NC, BR, D), lambda i: (0, i, 0)),
                  pl.BlockSpec((BR, NW), lambda i: (i, 0)),
                  pl.BlockSpec((BR, D), lambda i: (i, 0))],
        out_specs=pl.BlockSpec((BR, D), lambda i: (i, 0)),
        out_shape=jax.ShapeDtypeStruct((N, D), jnp.float32),
    )(s2, cntT, r2)


def kernel(x, edge_index, W1_l, b1_l, W1_r, W2_l, b2_l, W2_r):
    N, D = x.shape
    E = edge_index.shape[1]
    src = edge_index[0].astype(jnp.int32)
    dst = edge_index[1].astype(jnp.int32)
    Npad = -(-N // 128) * 128
    zeros = jnp.zeros((Npad, D), jnp.float32)

    cnt = _make_cnt(N, E)(dst)
    cntT = cnt.T
    y1, r1 = _tc_pre(x, W1_l, W1_r, b1_l)
    part1 = _make_agg(N, Npad, D, E)(y1, src, dst, zeros)
    y2, r2 = _tc_mid(part1, cntT, r1, W2_l, W2_r, b2_l)
    part2 = _make_agg(N, Npad, D, E)(y2, src, dst, zeros)
    return _tc_post(part2, cntT, r2)


# speedup vs baseline: 6.4794x; 1.2460x over previous
import jax
import jax.numpy as jnp
from jax import lax
from jax.experimental import pallas as pl
from jax.experimental.pallas import tpu as pltpu
from jax.experimental.pallas import tpu_sc as plsc

NC = 2
NS = 16
NW = NC * NS
LANES = 16
CHUNK = 80


def _make_agg(N, Npad, D, E, with_counts):
    ept = E // NW
    nchunk = ept // CHUNK
    rpt = Npad // NS
    mesh = plsc.VectorSubcoreMesh(core_axis_name="c", subcore_axis_name="s")

    out_type = [jax.ShapeDtypeStruct((NC, Npad, D), jnp.float32)]
    scratch = [
        pltpu.VMEM((CHUNK,), jnp.int32),
        pltpu.VMEM((CHUNK,), jnp.int32),
        pltpu.VMEM((CHUNK,), jnp.int32),
        pltpu.VMEM((CHUNK,), jnp.int32),
        pltpu.VMEM((CHUNK, D), jnp.float32),
        pltpu.VMEM((CHUNK, D), jnp.float32),
        pltpu.VMEM_SHARED((Npad, D), jnp.float32),
        pltpu.SemaphoreType.DMA,
        pltpu.SemaphoreType.DMA,
    ]
    if with_counts:
        out_type.append(jax.ShapeDtypeStruct((NW, N), jnp.float32))
        scratch.append(pltpu.VMEM((N,), jnp.float32))

    def body(y_hbm, src_hbm, dst_hbm, zeros_hbm, *rest):
        if with_counts:
            (out_hbm, cnt_hbm, idx0, idx1, didx0, didx1,
             rows0, rows1, acc, sem0, sem1, cnt_v) = rest
        else:
            (out_hbm, idx0, idx1, didx0, didx1,
             rows0, rows1, acc, sem0, sem1) = rest
        bufs = ((idx0, didx0, rows0, sem0), (idx1, didx1, rows1, sem1))
        cid = lax.axis_index("c")
        sid = lax.axis_index("s")
        wid = sid * NC + cid
        stripe = pl.ds(sid * rpt, rpt)
        base = wid * ept
        ones = jnp.ones((LANES,), jnp.float32)

        pltpu.sync_copy(zeros_hbm.at[stripe], acc.at[stripe])
        if with_counts:
            zv = jnp.zeros((LANES,), jnp.float32)

            def zbody(i, carry):
                cnt_v[pl.ds(i * LANES, LANES)] = zv
                return carry

            lax.fori_loop(0, N // LANES, zbody, 0)
        plsc.subcore_barrier()

        def load(b, c):
            off = pl.multiple_of(base + c * CHUNK, 8)
            pltpu.sync_copy(src_hbm.at[pl.ds(off, CHUNK)], bufs[b][0])
            pltpu.sync_copy(dst_hbm.at[pl.ds(off, CHUNK)], bufs[b][1])

        def start_gather(b):
            pltpu.async_copy(y_hbm.at[bufs[b][0]], bufs[b][2], bufs[b][3])

        def wait_gather(b):
            pltpu.make_async_copy(y_hbm.at[bufs[b][0]], bufs[b][2],
                                  bufs[b][3]).wait()

        def finish(b):
            if with_counts:
                for i in range(CHUNK // LANES):
                    iv = bufs[b][1][pl.ds(i * LANES, LANES)]
                    plsc.addupdate_scatter(cnt_v, [iv], ones)
            pltpu.sync_copy(bufs[b][2], acc.at[bufs[b][1]], add=True)

        def handle(b, c_cur):
            wait_gather(b)
            load(1 - b, c_cur + 1)
            start_gather(1 - b)
            finish(b)

        load(0, 0)
        start_gather(0)

        def pair_body(g, carry):
            handle(0, 2 * g)
            handle(1, 2 * g + 1)
            return carry

        lax.fori_loop(0, (nchunk - 1) // 2, pair_body, 0)
        wait_gather((nchunk - 1) % 2)
        finish((nchunk - 1) % 2)
        plsc.subcore_barrier()

        pltpu.sync_copy(acc.at[stripe], out_hbm.at[cid, stripe])
        if with_counts:
            pltpu.sync_copy(cnt_v, cnt_hbm.at[wid])

    params = pltpu.CompilerParams(needs_layout_passes=False) if with_counts \
        else None
    return pl.kernel(body, out_type=out_type, mesh=mesh,
                     scratch_types=scratch, compiler_params=params)


_DOT = (((1,), (1,)), ((), ()))


def _tc_pre(x, W_l, W_r, b):
    N, D = x.shape
    BR = 1000

    def body(x_ref, wl_ref, wr_ref, b_ref, y_ref, r_ref):
        xb = x_ref[...]
        y_ref[...] = lax.dot_general(xb, wl_ref[...], _DOT,
                                     preferred_element_type=jnp.float32)
        r_ref[...] = lax.dot_general(xb, wr_ref[...], _DOT,
                                     preferred_element_type=jnp.float32) + b_ref[...]

    return pl.pallas_call(
        body,
        grid=(N // BR,),
        in_specs=[pl.BlockSpec((BR, D), lambda i: (i, 0)),
                  pl.BlockSpec((D, D), lambda i: (0, 0)),
                  pl.BlockSpec((D, D), lambda i: (0, 0)),
                  pl.BlockSpec((1, D), lambda i: (0, 0))],
        out_specs=[pl.BlockSpec((BR, D), lambda i: (i, 0)),
                   pl.BlockSpec((BR, D), lambda i: (i, 0))],
        out_shape=[jax.ShapeDtypeStruct((N, D), jnp.float32)] * 2,
    )(x, W_l, W_r, b.reshape(1, D))


def _tc_mid(s1, cntT, r1, W2_l, W2_r, b2):
    N, D = r1.shape
    BR = 1000

    def body(s_ref, c_ref, r_ref, wl_ref, wr_ref, b_ref, y_ref, rr_ref):
        s = s_ref[0] + s_ref[1]
        c = jnp.sum(c_ref[...], axis=1, keepdims=True)
        pre = s / jnp.maximum(c, 1.0) + r_ref[...]
        nrm = jnp.sqrt(jnp.sum(pre * pre, axis=1, keepdims=True))
        h = jnp.maximum(pre / jnp.maximum(nrm, 1e-12), 0.0)
        y_ref[...] = lax.dot_general(h, wl_ref[...], _DOT,
                                     preferred_element_type=jnp.float32)
        rr_ref[...] = lax.dot_general(h, wr_ref[...], _DOT,
                                      preferred_element_type=jnp.float32) + b_ref[...]

    return pl.pallas_call(
        body,
        grid=(N // BR,),
        in_specs=[pl.BlockSpec((NC, BR, D), lambda i: (0, i, 0)),
                  pl.BlockSpec((BR, NW), lambda i: (i, 0)),
                  pl.BlockSpec((BR, D), lambda i: (i, 0)),
                  pl.BlockSpec((D, D), lambda i: (0, 0)),
                  pl.BlockSpec((D, D), lambda i: (0, 0)),
                  pl.BlockSpec((1, D), lambda i: (0, 0))],
        out_specs=[pl.BlockSpec((BR, D), lambda i: (i, 0)),
                   pl.BlockSpec((BR, D), lambda i: (i, 0))],
        out_shape=[jax.ShapeDtypeStruct((N, D), jnp.float32)] * 2,
    )(s1, cntT, r1, W2_l, W2_r, b2.reshape(1, D))


def _tc_post(s2, cntT, r2):
    N, D = r2.shape
    BR = 1000

    def body(s_ref, c_ref, r_ref, o_ref):
        s = s_ref[0] + s_ref[1]
        c = jnp.sum(c_ref[...], axis=1, keepdims=True)
        o_ref[...] = s / jnp.maximum(c, 1.0) + r_ref[...]

    return pl.pallas_call(
        body,
        grid=(N // BR,),
        in_specs=[pl.BlockSpec((NC, BR, D), lambda i: (0, i, 0)),
                  pl.BlockSpec((BR, NW), lambda i: (i, 0)),
                  pl.BlockSpec((BR, D), lambda i: (i, 0))],
        out_specs=pl.BlockSpec((BR, D), lambda i: (i, 0)),
        out_shape=jax.ShapeDtypeStruct((N, D), jnp.float32),
    )(s2, cntT, r2)


def kernel(x, edge_index, W1_l, b1_l, W1_r, W2_l, b2_l, W2_r):
    N, D = x.shape
    E = edge_index.shape[1]
    src = edge_index[0].astype(jnp.int32)
    dst = edge_index[1].astype(jnp.int32)
    Npad = -(-N // 128) * 128
    zeros = jnp.zeros((Npad, D), jnp.float32)

    y1, r1 = _tc_pre(x, W1_l, W1_r, b1_l)
    part1, cnt = _make_agg(N, Npad, D, E, True)(y1, src, dst, zeros)
    cntT = cnt.T
    y2, r2 = _tc_mid(part1, cntT, r1, W2_l, W2_r, b2_l)
    part2, = _make_agg(N, Npad, D, E, False)(y2, src, dst, zeros)
    return _tc_post(part2, cntT, r2)


# speedup vs baseline: 10.3524x; 1.5977x over previous
import jax
import jax.numpy as jnp
from jax import lax
from jax.experimental import pallas as pl
from jax.experimental.pallas import tpu as pltpu
from jax.experimental.pallas import tpu_sc as plsc

NC = 2
NS = 16
NW = NC * NS
LANES = 16
CHUNK = 80


def _make_agg(N, Npad, D, E, with_counts):
    ept = E // NW
    nchunk = ept // CHUNK
    rpt = Npad // NS
    mesh = plsc.VectorSubcoreMesh(core_axis_name="c", subcore_axis_name="s")

    out_type = [jax.ShapeDtypeStruct((NC, Npad, D), jnp.float32)]
    scratch = [
        pltpu.VMEM((CHUNK,), jnp.int32),
        pltpu.VMEM((CHUNK,), jnp.int32),
        pltpu.VMEM((CHUNK,), jnp.int32),
        pltpu.VMEM((CHUNK,), jnp.int32),
        pltpu.VMEM((CHUNK, D), jnp.float32),
        pltpu.VMEM((CHUNK, D), jnp.float32),
        pltpu.VMEM_SHARED((Npad, D), jnp.float32),
        pltpu.SemaphoreType.DMA,
        pltpu.SemaphoreType.DMA,
        pltpu.SemaphoreType.DMA,
        pltpu.SemaphoreType.DMA,
        pltpu.SemaphoreType.DMA,
        pltpu.SemaphoreType.DMA,
    ]
    if with_counts:
        out_type.append(jax.ShapeDtypeStruct((NW, N), jnp.float32))
        scratch.append(pltpu.VMEM((N,), jnp.float32))

    def body(y_hbm, src_hbm, zeros_hbm, *rest):
        if with_counts:
            (out_hbm, cnt_hbm, idx0, idx1, didx0, didx1, rows0, rows1, acc,
             semg0, semg1, semi0, semi1, semd0, semd1, cnt_v) = rest
        else:
            (out_hbm, idx0, idx1, didx0, didx1, rows0, rows1, acc,
             semg0, semg1, semi0, semi1, semd0, semd1) = rest
        idxs = (idx0, idx1)
        didxs = (didx0, didx1)
        rows = (rows0, rows1)
        semg = (semg0, semg1)
        semi = (semi0, semi1)
        semd = (semd0, semd1)
        cid = lax.axis_index("c")
        sid = lax.axis_index("s")
        wid = sid * NC + cid
        stripe = pl.ds(sid * rpt, rpt)
        base = wid * ept
        ones = jnp.ones((LANES,), jnp.float32)

        pltpu.sync_copy(zeros_hbm.at[stripe], acc.at[stripe])
        if with_counts:
            zv = jnp.zeros((LANES,), jnp.float32)

            def zbody(i, carry):
                cnt_v[pl.ds(i * LANES, LANES)] = zv
                return carry

            lax.fori_loop(0, N // LANES, zbody, 0)
        plsc.subcore_barrier()

        def idx_descr(b, c):
            off = pl.multiple_of(base + c * CHUNK, 8)
            return (pltpu.make_async_copy(src_hbm.at[pl.ds(off, CHUNK)],
                                          idxs[b], semi[b]),
                    pltpu.make_async_copy(src_hbm.at[pl.ds(E + off, CHUNK)],
                                          didxs[b], semd[b]))

        def start_idx(b, c):
            for d in idx_descr(b, c):
                d.start()

        def wait_idx(b, c):
            for d in idx_descr(b, c):
                d.wait()

        def gather_descr(b):
            return pltpu.make_async_copy(y_hbm.at[idxs[b]], rows[b], semg[b])

        def finish(b):
            if with_counts:
                for i in range(CHUNK // LANES):
                    iv = didxs[b][pl.ds(i * LANES, LANES)]
                    plsc.addupdate_scatter(cnt_v, [iv], ones)
            pltpu.sync_copy(rows[b], acc.at[didxs[b]], add=True)

        def handle(b, c):
            wait_idx(1 - b, c + 1)
            gather_descr(1 - b).start()
            gather_descr(b).wait()
            finish(b)
            start_idx(b, c + 2)

        start_idx(0, 0)
        wait_idx(0, 0)
        gather_descr(0).start()
        start_idx(1, 1)

        def pair_body(g, carry):
            handle(0, 2 * g)
            handle(1, 2 * g + 1)
            return carry

        lax.fori_loop(0, (nchunk - 1) // 2, pair_body, 0)
        wait_idx(1, nchunk)
        gather_descr(0).wait()
        finish(0)
        plsc.subcore_barrier()

        pltpu.sync_copy(acc.at[stripe], out_hbm.at[cid, stripe])
        if with_counts:
            pltpu.sync_copy(cnt_v, cnt_hbm.at[wid])

    params = pltpu.CompilerParams(needs_layout_passes=False) if with_counts \
        else None
    return pl.kernel(body, out_type=out_type, mesh=mesh,
                     scratch_types=scratch, compiler_params=params)


_DOT = (((1,), (1,)), ((), ()))


def _tc_pre(x, W_l, W_r, b):
    N, D = x.shape
    BR = 1000

    def body(x_ref, wl_ref, wr_ref, b_ref, y_ref, r_ref):
        xb = x_ref[...]
        y_ref[...] = lax.dot_general(xb, wl_ref[...], _DOT,
                                     preferred_element_type=jnp.float32)
        r_ref[...] = lax.dot_general(xb, wr_ref[...], _DOT,
                                     preferred_element_type=jnp.float32) + b_ref[...]

    return pl.pallas_call(
        body,
        grid=(N // BR,),
        in_specs=[pl.BlockSpec((BR, D), lambda i: (i, 0)),
                  pl.BlockSpec((D, D), lambda i: (0, 0)),
                  pl.BlockSpec((D, D), lambda i: (0, 0)),
                  pl.BlockSpec((1, D), lambda i: (0, 0))],
        out_specs=[pl.BlockSpec((BR, D), lambda i: (i, 0)),
                   pl.BlockSpec((BR, D), lambda i: (i, 0))],
        out_shape=[jax.ShapeDtypeStruct((N, D), jnp.float32)] * 2,
    )(x, W_l, W_r, b.reshape(1, D))


def _tc_mid(s1, cntT, r1, W2_l, W2_r, b2):
    N, D = r1.shape
    BR = 1000

    def body(s_ref, c_ref, r_ref, wl_ref, wr_ref, b_ref, y_ref, rr_ref):
        s = s_ref[0] + s_ref[1]
        c = jnp.sum(c_ref[...], axis=1, keepdims=True)
        pre = s / jnp.maximum(c, 1.0) + r_ref[...]
        nrm = jnp.sqrt(jnp.sum(pre * pre, axis=1, keepdims=True))
        h = jnp.maximum(pre / jnp.maximum(nrm, 1e-12), 0.0)
        y_ref[...] = lax.dot_general(h, wl_ref[...], _DOT,
                                     preferred_element_type=jnp.float32)
        rr_ref[...] = lax.dot_general(h, wr_ref[...], _DOT,
                                      preferred_element_type=jnp.float32) + b_ref[...]

    return pl.pallas_call(
        body,
        grid=(N // BR,),
        in_specs=[pl.BlockSpec((NC, BR, D), lambda i: (0, i, 0)),
                  pl.BlockSpec((BR, NW), lambda i: (i, 0)),
                  pl.BlockSpec((BR, D), lambda i: (i, 0)),
                  pl.BlockSpec((D, D), lambda i: (0, 0)),
                  pl.BlockSpec((D, D), lambda i: (0, 0)),
                  pl.BlockSpec((1, D), lambda i: (0, 0))],
        out_specs=[pl.BlockSpec((BR, D), lambda i: (i, 0)),
                   pl.BlockSpec((BR, D), lambda i: (i, 0))],
        out_shape=[jax.ShapeDtypeStruct((N, D), jnp.float32)] * 2,
    )(s1, cntT, r1, W2_l, W2_r, b2.reshape(1, D))


def _tc_post(s2, cntT, r2):
    N, D = r2.shape
    BR = 1000

    def body(s_ref, c_ref, r_ref, o_ref):
        s = s_ref[0] + s_ref[1]
        c = jnp.sum(c_ref[...], axis=1, keepdims=True)
        o_ref[...] = s / jnp.maximum(c, 1.0) + r_ref[...]

    return pl.pallas_call(
        body,
        grid=(N // BR,),
        in_specs=[pl.BlockSpec((NC, BR, D), lambda i: (0, i, 0)),
                  pl.BlockSpec((BR, NW), lambda i: (i, 0)),
                  pl.BlockSpec((BR, D), lambda i: (i, 0))],
        out_specs=pl.BlockSpec((BR, D), lambda i: (i, 0)),
        out_shape=jax.ShapeDtypeStruct((N, D), jnp.float32),
    )(s2, cntT, r2)


def kernel(x, edge_index, W1_l, b1_l, W1_r, W2_l, b2_l, W2_r):
    N, D = x.shape
    E = edge_index.shape[1]
    src = edge_index[0].astype(jnp.int32)
    dst = edge_index[1].astype(jnp.int32)
    srcdst = jnp.concatenate([src, dst, jnp.zeros((CHUNK,), jnp.int32)])
    Npad = -(-N // 128) * 128
    zeros = jnp.zeros((Npad, D), jnp.float32)

    y1, r1 = _tc_pre(x, W1_l, W1_r, b1_l)
    part1, cnt = _make_agg(N, Npad, D, E, True)(y1, srcdst, zeros)
    cntT = cnt.T
    y2, r2 = _tc_mid(part1, cntT, r1, W2_l, W2_r, b2_l)
    part2, = _make_agg(N, Npad, D, E, False)(y2, srcdst, zeros)
    return _tc_post(part2, cntT, r2)


# speedup vs baseline: 13.6826x; 1.3217x over previous
import jax
import jax.numpy as jnp
from jax import lax
from jax.experimental import pallas as pl
from jax.experimental.pallas import tpu as pltpu
from jax.experimental.pallas import tpu_sc as plsc

NC = 2
NS = 16
NW = NC * NS
LANES = 16
CHUNK = 80


def _make_agg(N, Npad, D, E, with_counts):
    ept = E // NW
    nchunk = ept // CHUNK
    rpt = Npad // NS
    mesh = plsc.VectorSubcoreMesh(core_axis_name="c", subcore_axis_name="s")

    NR = 3
    NI = 6
    out_type = [jax.ShapeDtypeStruct((NC, Npad, D), jnp.float32)]
    scratch = (
        [pltpu.VMEM((CHUNK,), jnp.int32) for _ in range(NI)]
        + [pltpu.VMEM((CHUNK,), jnp.int32) for _ in range(NI)]
        + [pltpu.VMEM((CHUNK, D), jnp.float32) for _ in range(NR)]
        + [pltpu.VMEM_SHARED((Npad, D), jnp.float32)]
        + [pltpu.SemaphoreType.DMA] * (2 * NR + 2 * NI)
    )
    if with_counts:
        out_type.append(jax.ShapeDtypeStruct((NW, N), jnp.float32))
        scratch.append(pltpu.VMEM((N,), jnp.float32))

    def body(y_hbm, src_hbm, zeros_hbm, *rest):
        out_hbm = rest[0]
        rest = list(rest[1:])
        if with_counts:
            cnt_hbm = rest.pop(0)
            cnt_v = rest.pop(-1)
        idxs = rest[0:NI]
        didxs = rest[NI:2 * NI]
        rows = rest[2 * NI:2 * NI + NR]
        acc = rest[2 * NI + NR]
        sems = rest[2 * NI + NR + 1:]
        semg = sems[0:NR]
        semsc = sems[NR:2 * NR]
        semi = sems[2 * NR:2 * NR + NI]
        semd = sems[2 * NR + NI:]
        cid = lax.axis_index("c")
        sid = lax.axis_index("s")
        wid = sid * NC + cid
        stripe = pl.ds(sid * rpt, rpt)
        base = wid * ept
        ones = jnp.ones((LANES,), jnp.float32)

        pltpu.sync_copy(zeros_hbm.at[stripe], acc.at[stripe])
        if with_counts:
            zv = jnp.zeros((LANES,), jnp.float32)

            def zbody(i, carry):
                cnt_v[pl.ds(i * LANES, LANES)] = zv
                return carry

            lax.fori_loop(0, N // LANES, zbody, 0)
        plsc.subcore_barrier()

        def idx_descr(b, c):
            off = pl.multiple_of(base + c * CHUNK, 8)
            return (pltpu.make_async_copy(src_hbm.at[pl.ds(off, CHUNK)],
                                          idxs[b], semi[b]),
                    pltpu.make_async_copy(src_hbm.at[pl.ds(E + off, CHUNK)],
                                          didxs[b], semd[b]))

        def gather_descr(rb, ib):
            return pltpu.make_async_copy(y_hbm.at[idxs[ib]], rows[rb],
                                         semg[rb])

        def scatter_start(rb, ib):
            pltpu.async_copy(rows[rb], acc.at[didxs[ib]], semsc[rb], add=True)

        def scatter_wait(rb, ib):
            pltpu.make_async_copy(rows[rb], acc.at[didxs[ib]],
                                  semsc[rb]).wait()

        def counts(ib):
            if with_counts:
                for i in range(CHUNK // LANES):
                    iv = didxs[ib][pl.ds(i * LANES, LANES)]
                    plsc.addupdate_scatter(cnt_v, [iv], ones)

        def steady(c, u, drain, gather, idx):
            rb, ib = u % NR, u % NI
            gather_descr(rb, ib).wait()
            scatter_start(rb, ib)
            counts(ib)
            if drain:
                scatter_wait((u + 2) % NR, (u + 5) % NI)
            if gather:
                rb2, ib2 = (u + 2) % NR, (u + 2) % NI
                for d in idx_descr(ib2, c + 2):
                    d.wait()
                gather_descr(rb2, ib2).start()
            if idx:
                for d in idx_descr((u + 4) % NI, c + 4):
                    d.start()

        for c in range(4):
            for d in idx_descr(c % NI, c):
                d.start()
        for c in range(2):
            for d in idx_descr(c % NI, c):
                d.wait()
            gather_descr(c % NR, c % NI).start()

        for c in range(6):
            steady(c, c, drain=(c > 0), gather=True, idx=(c + 4 < nchunk))

        G = (nchunk - 6 - 5) // 6

        def group_body(g, carry):
            c0 = 6 * g
            for u in range(6):
                steady(c0 + u, u, drain=True, gather=True, idx=True)
            return carry

        lax.fori_loop(1, G + 1, group_body, 0)

        for c in range(6 * (G + 1), nchunk):
            steady(c, c % 6, drain=True,
                   gather=(c + 2 < nchunk), idx=(c + 4 < nchunk))
        scatter_wait((nchunk - 1) % NR, (nchunk - 1) % NI)
        plsc.subcore_barrier()

        pltpu.sync_copy(acc.at[stripe], out_hbm.at[cid, stripe])
        if with_counts:
            pltpu.sync_copy(cnt_v, cnt_hbm.at[wid])

    params = pltpu.CompilerParams(needs_layout_passes=False) if with_counts \
        else None
    return pl.kernel(body, out_type=out_type, mesh=mesh,
                     scratch_types=scratch, compiler_params=params)


_DOT = (((1,), (1,)), ((), ()))


def _tc_pre(x, W_l, W_r, b):
    N, D = x.shape
    BR = 1000

    def body(x_ref, wl_ref, wr_ref, b_ref, y_ref, r_ref):
        xb = x_ref[...]
        y_ref[...] = lax.dot_general(xb, wl_ref[...], _DOT,
                                     preferred_element_type=jnp.float32)
        r_ref[...] = lax.dot_general(xb, wr_ref[...], _DOT,
                                     preferred_element_type=jnp.float32) + b_ref[...]

    return pl.pallas_call(
        body,
        grid=(N // BR,),
        in_specs=[pl.BlockSpec((BR, D), lambda i: (i, 0)),
                  pl.BlockSpec((D, D), lambda i: (0, 0)),
                  pl.BlockSpec((D, D), lambda i: (0, 0)),
                  pl.BlockSpec((1, D), lambda i: (0, 0))],
        out_specs=[pl.BlockSpec((BR, D), lambda i: (i, 0)),
                   pl.BlockSpec((BR, D), lambda i: (i, 0))],
        out_shape=[jax.ShapeDtypeStruct((N, D), jnp.float32)] * 2,
    )(x, W_l, W_r, b.reshape(1, D))


def _tc_mid(s1, cntT, r1, W2_l, W2_r, b2):
    N, D = r1.shape
    BR = 1000

    def body(s_ref, c_ref, r_ref, wl_ref, wr_ref, b_ref, y_ref, rr_ref):
        s = s_ref[0] + s_ref[1]
        c = jnp.sum(c_ref[...], axis=1, keepdims=True)
        pre = s / jnp.maximum(c, 1.0) + r_ref[...]
        nrm = jnp.sqrt(jnp.sum(pre * pre, axis=1, keepdims=True))
        h = jnp.maximum(pre / jnp.maximum(nrm, 1e-12), 0.0)
        y_ref[...] = lax.dot_general(h, wl_ref[...], _DOT,
                                     preferred_element_type=jnp.float32)
        rr_ref[...] = lax.dot_general(h, wr_ref[...], _DOT,
                                      preferred_element_type=jnp.float32) + b_ref[...]

    return pl.pallas_call(
        body,
        grid=(N // BR,),
        in_specs=[pl.BlockSpec((NC, BR, D), lambda i: (0, i, 0)),
                  pl.BlockSpec((BR, NW), lambda i: (i, 0)),
                  pl.BlockSpec((BR, D), lambda i: (i, 0)),
                  pl.BlockSpec((D, D), lambda i: (0, 0)),
                  pl.BlockSpec((D, D), lambda i: (0, 0)),
                  pl.BlockSpec((1, D), lambda i: (0, 0))],
        out_specs=[pl.BlockSpec((BR, D), lambda i: (i, 0)),
                   pl.BlockSpec((BR, D), lambda i: (i, 0))],
        out_shape=[jax.ShapeDtypeStruct((N, D), jnp.float32)] * 2,
    )(s1, cntT, r1, W2_l, W2_r, b2.reshape(1, D))


def _tc_post(s2, cntT, r2):
    N, D = r2.shape
    BR = 1000

    def body(s_ref, c_ref, r_ref, o_ref):
        s = s_ref[0] + s_ref[1]
        c = jnp.sum(c_ref[...], axis=1, keepdims=True)
        o_ref[...] = s / jnp.maximum(c, 1.0) + r_ref[...]

    return pl.pallas_call(
        body,
        grid=(N // BR,),
        in_specs=[pl.BlockSpec((NC, BR, D), lambda i: (0, i, 0)),
                  pl.BlockSpec((BR, NW), lambda i: (i, 0)),
                  pl.BlockSpec((BR, D), lambda i: (i, 0))],
        out_specs=pl.BlockSpec((BR, D), lambda i: (i, 0)),
        out_shape=jax.ShapeDtypeStruct((N, D), jnp.float32),
    )(s2, cntT, r2)


def kernel(x, edge_index, W1_l, b1_l, W1_r, W2_l, b2_l, W2_r):
    N, D = x.shape
    E = edge_index.shape[1]
    src = edge_index[0].astype(jnp.int32)
    dst = edge_index[1].astype(jnp.int32)
    srcdst = jnp.concatenate([src, dst, jnp.zeros((CHUNK,), jnp.int32)])
    Npad = -(-N // 128) * 128
    zeros = jnp.zeros((Npad, D), jnp.float32)

    y1, r1 = _tc_pre(x, W1_l, W1_r, b1_l)
    part1, cnt = _make_agg(N, Npad, D, E, True)(y1, srcdst, zeros)
    cntT = cnt.T
    y2, r2 = _tc_mid(part1, cntT, r1, W2_l, W2_r, b2_l)
    part2, = _make_agg(N, Npad, D, E, False)(y2, srcdst, zeros)
    return _tc_post(part2, cntT, r2)


# speedup vs baseline: 14.2658x; 1.0426x over previous
import jax
import jax.numpy as jnp
from jax import lax
from jax.experimental import pallas as pl
from jax.experimental.pallas import tpu as pltpu
from jax.experimental.pallas import tpu_sc as plsc

NC = 2
NS = 16
NW = NC * NS
LANES = 16
CHUNK = 80


def _make_agg(N, Npad, D, E, with_counts):
    ept = E // NW
    nchunk = ept // CHUNK
    rpt = Npad // NS
    mesh = plsc.VectorSubcoreMesh(core_axis_name="c", subcore_axis_name="s")

    NR = 3
    NI = 6
    out_type = [jax.ShapeDtypeStruct((NC, Npad, D), jnp.float32)]
    scratch = (
        [pltpu.VMEM((CHUNK,), jnp.int32) for _ in range(NI)]
        + [pltpu.VMEM((CHUNK,), jnp.int32) for _ in range(NI)]
        + [pltpu.VMEM((CHUNK, D), jnp.float32) for _ in range(NR)]
        + [pltpu.VMEM_SHARED((Npad, D), jnp.float32)]
        + [pltpu.SemaphoreType.DMA] * (2 * NR + 2 * NI)
    )
    if with_counts:
        out_type.append(jax.ShapeDtypeStruct((NW, N), jnp.float32))
        scratch.append(pltpu.VMEM((N,), jnp.float32))

    def body(y_hbm, src_hbm, dst_hbm, zeros_hbm, *rest):
        out_hbm = rest[0]
        rest = list(rest[1:])
        if with_counts:
            cnt_hbm = rest.pop(0)
            cnt_v = rest.pop(-1)
        idxs = rest[0:NI]
        didxs = rest[NI:2 * NI]
        rows = rest[2 * NI:2 * NI + NR]
        acc = rest[2 * NI + NR]
        sems = rest[2 * NI + NR + 1:]
        semg = sems[0:NR]
        semsc = sems[NR:2 * NR]
        semi = sems[2 * NR:2 * NR + NI]
        semd = sems[2 * NR + NI:]
        cid = lax.axis_index("c")
        sid = lax.axis_index("s")
        wid = sid * NC + cid
        stripe = pl.ds(sid * rpt, rpt)
        base = wid * ept
        ones = jnp.ones((LANES,), jnp.float32)

        pltpu.sync_copy(zeros_hbm.at[stripe], acc.at[stripe])
        if with_counts:
            zv = jnp.zeros((LANES,), jnp.float32)

            def zbody(i, carry):
                cnt_v[pl.ds(i * LANES, LANES)] = zv
                return carry

            lax.fori_loop(0, N // LANES, zbody, 0)
        plsc.subcore_barrier()

        def idx_descr(b, c):
            off = pl.multiple_of(base + c * CHUNK, 8)
            return (pltpu.make_async_copy(src_hbm.at[pl.ds(off, CHUNK)],
                                          idxs[b], semi[b]),
                    pltpu.make_async_copy(dst_hbm.at[pl.ds(off, CHUNK)],
                                          didxs[b], semd[b]))

        def gather_descr(rb, ib):
            return pltpu.make_async_copy(y_hbm.at[idxs[ib]], rows[rb],
                                         semg[rb])

        def scatter_start(rb, ib):
            pltpu.async_copy(rows[rb], acc.at[didxs[ib]], semsc[rb], add=True)

        def scatter_wait(rb, ib):
            pltpu.make_async_copy(rows[rb], acc.at[didxs[ib]],
                                  semsc[rb]).wait()

        def counts(ib):
            if with_counts:
                for i in range(CHUNK // LANES):
                    iv = didxs[ib][pl.ds(i * LANES, LANES)]
                    plsc.addupdate_scatter(cnt_v, [iv], ones)

        def steady(c, u, drain, gather, idx):
            rb, ib = u % NR, u % NI
            gather_descr(rb, ib).wait()
            scatter_start(rb, ib)
            counts(ib)
            if drain:
                scatter_wait((u + 2) % NR, (u + 5) % NI)
            if gather:
                rb2, ib2 = (u + 2) % NR, (u + 2) % NI
                for d in idx_descr(ib2, c + 2):
                    d.wait()
                gather_descr(rb2, ib2).start()
            if idx:
                for d in idx_descr((u + 4) % NI, c + 4):
                    d.start()

        for c in range(4):
            for d in idx_descr(c % NI, c):
                d.start()
        for c in range(2):
            for d in idx_descr(c % NI, c):
                d.wait()
            gather_descr(c % NR, c % NI).start()

        for c in range(6):
            steady(c, c, drain=(c > 0), gather=True, idx=(c + 4 < nchunk))

        G = (nchunk - 6 - 5) // 6

        def group_body(g, carry):
            c0 = 6 * g
            for u in range(6):
                steady(c0 + u, u, drain=True, gather=True, idx=True)
            return carry

        lax.fori_loop(1, G + 1, group_body, 0)

        for c in range(6 * (G + 1), nchunk):
            steady(c, c % 6, drain=True,
                   gather=(c + 2 < nchunk), idx=(c + 4 < nchunk))
        scatter_wait((nchunk - 1) % NR, (nchunk - 1) % NI)
        plsc.subcore_barrier()

        pltpu.sync_copy(acc.at[stripe], out_hbm.at[cid, stripe])
        if with_counts:
            pltpu.sync_copy(cnt_v, cnt_hbm.at[wid])

    params = pltpu.CompilerParams(needs_layout_passes=False) if with_counts \
        else None
    return pl.kernel(body, out_type=out_type, mesh=mesh,
                     scratch_types=scratch, compiler_params=params)


_DOT = (((1,), (1,)), ((), ()))


def _tc_pre(x, W_r):
    N, D = x.shape
    BR = 1000

    def body(x_ref, wr_ref, r_ref):
        r_ref[...] = lax.dot_general(x_ref[...], wr_ref[...], _DOT,
                                     preferred_element_type=jnp.float32)

    return pl.pallas_call(
        body,
        grid=(N // BR,),
        in_specs=[pl.BlockSpec((BR, D), lambda i: (i, 0)),
                  pl.BlockSpec((D, D), lambda i: (0, 0))],
        out_specs=pl.BlockSpec((BR, D), lambda i: (i, 0)),
        out_shape=jax.ShapeDtypeStruct((N, D), jnp.float32),
    )(x, W_r)


def _tc_mid(s1, cntT, r1, W1_l, b1, W2_r):
    N, D = r1.shape
    BR = 1000

    def body(s_ref, c_ref, r_ref, wl_ref, b_ref, wr_ref, h_ref, rr_ref):
        s = s_ref[0] + s_ref[1]
        t = lax.dot_general(s, wl_ref[...], _DOT,
                            preferred_element_type=jnp.float32)
        c = jnp.sum(c_ref[...], axis=1, keepdims=True)
        pre = t / jnp.maximum(c, 1.0) + b_ref[...] + r_ref[...]
        nrm = jnp.sqrt(jnp.sum(pre * pre, axis=1, keepdims=True))
        h = jnp.maximum(pre / jnp.maximum(nrm, 1e-12), 0.0)
        h_ref[...] = h
        rr_ref[...] = lax.dot_general(h, wr_ref[...], _DOT,
                                      preferred_element_type=jnp.float32)

    return pl.pallas_call(
        body,
        grid=(N // BR,),
        in_specs=[pl.BlockSpec((NC, BR, D), lambda i: (0, i, 0)),
                  pl.BlockSpec((BR, NW), lambda i: (i, 0)),
                  pl.BlockSpec((BR, D), lambda i: (i, 0)),
                  pl.BlockSpec((D, D), lambda i: (0, 0)),
                  pl.BlockSpec((1, D), lambda i: (0, 0)),
                  pl.BlockSpec((D, D), lambda i: (0, 0))],
        out_specs=[pl.BlockSpec((BR, D), lambda i: (i, 0)),
                   pl.BlockSpec((BR, D), lambda i: (i, 0))],
        out_shape=[jax.ShapeDtypeStruct((N, D), jnp.float32)] * 2,
    )(s1, cntT, r1, W1_l, b1.reshape(1, D), W2_r)


def _tc_post(s2, cntT, r2, W2_l, b2):
    N, D = r2.shape
    BR = 1000

    def body(s_ref, c_ref, r_ref, wl_ref, b_ref, o_ref):
        s = s_ref[0] + s_ref[1]
        t = lax.dot_general(s, wl_ref[...], _DOT,
                            preferred_element_type=jnp.float32)
        c = jnp.sum(c_ref[...], axis=1, keepdims=True)
        o_ref[...] = t / jnp.maximum(c, 1.0) + b_ref[...] + r_ref[...]

    return pl.pallas_call(
        body,
        grid=(N // BR,),
        in_specs=[pl.BlockSpec((NC, BR, D), lambda i: (0, i, 0)),
                  pl.BlockSpec((BR, NW), lambda i: (i, 0)),
                  pl.BlockSpec((BR, D), lambda i: (i, 0)),
                  pl.BlockSpec((D, D), lambda i: (0, 0)),
                  pl.BlockSpec((1, D), lambda i: (0, 0))],
        out_specs=pl.BlockSpec((BR, D), lambda i: (i, 0)),
        out_shape=jax.ShapeDtypeStruct((N, D), jnp.float32),
    )(s2, cntT, r2, W2_l, b2.reshape(1, D))


def kernel(x, edge_index, W1_l, b1_l, W1_r, W2_l, b2_l, W2_r):
    N, D = x.shape
    E = edge_index.shape[1]
    src = edge_index[0].astype(jnp.int32)
    dst = edge_index[1].astype(jnp.int32)
    Npad = -(-N // 128) * 128
    zeros = jnp.zeros((Npad, D), jnp.float32)

    part1, cnt = _make_agg(N, Npad, D, E, True)(x, src, dst, zeros)
    r1 = _tc_pre(x, W1_r)
    cntT = cnt.T
    h, r2 = _tc_mid(part1, cntT, r1, W1_l, b1_l, W2_r)
    part2, = _make_agg(N, Npad, D, E, False)(h, src, dst, zeros)
    return _tc_post(part2, cntT, r2, W2_l, b2_l)


# speedup vs baseline: 14.3518x; 1.0060x over previous
import jax
import jax.numpy as jnp
from jax import lax
from jax.experimental import pallas as pl
from jax.experimental.pallas import tpu as pltpu
from jax.experimental.pallas import tpu_sc as plsc

NC = 2
NS = 16
NW = NC * NS
LANES = 16
CHUNK = 80


def _make_agg(N, Npad, D, E, with_counts):
    ept = E // NW
    nchunk = ept // CHUNK
    rpt = Npad // NS
    mesh = plsc.VectorSubcoreMesh(core_axis_name="c", subcore_axis_name="s")

    NR = 3
    NI = 6
    out_type = [jax.ShapeDtypeStruct((NC, Npad, D), jnp.float32)]
    scratch = (
        [pltpu.VMEM((CHUNK,), jnp.int32) for _ in range(NI)]
        + [pltpu.VMEM((CHUNK,), jnp.int32) for _ in range(NI)]
        + [pltpu.VMEM((CHUNK, D), jnp.float32) for _ in range(NR)]
        + [pltpu.VMEM_SHARED((Npad, D), jnp.float32)]
        + [pltpu.SemaphoreType.DMA] * (2 * NR + 2 * NI)
    )
    if with_counts:
        out_type.append(jax.ShapeDtypeStruct((NW, N), jnp.float32))
        scratch.append(pltpu.VMEM((N,), jnp.float32))

    def body(y_hbm, src_hbm, dst_hbm, zeros_hbm, *rest):
        out_hbm = rest[0]
        rest = list(rest[1:])
        if with_counts:
            cnt_hbm = rest.pop(0)
            cnt_v = rest.pop(-1)
        idxs = rest[0:NI]
        didxs = rest[NI:2 * NI]
        rows = rest[2 * NI:2 * NI + NR]
        acc = rest[2 * NI + NR]
        sems = rest[2 * NI + NR + 1:]
        semg = sems[0:NR]
        semsc = sems[NR:2 * NR]
        semi = sems[2 * NR:2 * NR + NI]
        semd = sems[2 * NR + NI:]
        cid = lax.axis_index("c")
        sid = lax.axis_index("s")
        wid = sid * NC + cid
        stripe = pl.ds(sid * rpt, rpt)
        base = wid * ept
        ones = jnp.ones((LANES,), jnp.float32)

        pltpu.sync_copy(zeros_hbm, acc.at[stripe])
        if with_counts:
            zv = jnp.zeros((LANES,), jnp.float32)

            def zbody(i, carry):
                cnt_v[pl.ds(i * LANES, LANES)] = zv
                return carry

            lax.fori_loop(0, N // LANES, zbody, 0)
        plsc.subcore_barrier()

        def idx_descr(b, c):
            off = pl.multiple_of(base + c * CHUNK, 8)
            return (pltpu.make_async_copy(src_hbm.at[pl.ds(off, CHUNK)],
                                          idxs[b], semi[b]),
                    pltpu.make_async_copy(dst_hbm.at[pl.ds(off, CHUNK)],
                                          didxs[b], semd[b]))

        def gather_descr(rb, ib):
            return pltpu.make_async_copy(y_hbm.at[idxs[ib]], rows[rb],
                                         semg[rb])

        def scatter_start(rb, ib):
            pltpu.async_copy(rows[rb], acc.at[didxs[ib]], semsc[rb], add=True)

        def scatter_wait(rb, ib):
            pltpu.make_async_copy(rows[rb], acc.at[didxs[ib]],
                                  semsc[rb]).wait()

        def counts(ib):
            if with_counts:
                for i in range(CHUNK // LANES):
                    iv = didxs[ib][pl.ds(i * LANES, LANES)]
                    plsc.addupdate_scatter(cnt_v, [iv], ones)

        def steady(c, u, drain, gather, idx):
            rb, ib = u % NR, u % NI
            gather_descr(rb, ib).wait()
            scatter_start(rb, ib)
            counts(ib)
            if drain:
                scatter_wait((u + 2) % NR, (u + 5) % NI)
            if gather:
                rb2, ib2 = (u + 2) % NR, (u + 2) % NI
                for d in idx_descr(ib2, c + 2):
                    d.wait()
                gather_descr(rb2, ib2).start()
            if idx:
                for d in idx_descr((u + 4) % NI, c + 4):
                    d.start()

        for c in range(4):
            for d in idx_descr(c % NI, c):
                d.start()
        for c in range(2):
            for d in idx_descr(c % NI, c):
                d.wait()
            gather_descr(c % NR, c % NI).start()

        for c in range(6):
            steady(c, c, drain=(c > 0), gather=True, idx=(c + 4 < nchunk))

        G = (nchunk - 6 - 5) // 6

        def group_body(g, carry):
            c0 = 6 * g
            for u in range(6):
                steady(c0 + u, u, drain=True, gather=True, idx=True)
            return carry

        lax.fori_loop(1, G + 1, group_body, 0)

        for c in range(6 * (G + 1), nchunk):
            steady(c, c % 6, drain=True,
                   gather=(c + 2 < nchunk), idx=(c + 4 < nchunk))
        scatter_wait((nchunk - 1) % NR, (nchunk - 1) % NI)
        plsc.subcore_barrier()

        pltpu.sync_copy(acc.at[stripe], out_hbm.at[cid, stripe])
        if with_counts:
            pltpu.sync_copy(cnt_v, cnt_hbm.at[wid])

    params = pltpu.CompilerParams(needs_layout_passes=False) if with_counts \
        else None
    return pl.kernel(body, out_type=out_type, mesh=mesh,
                     scratch_types=scratch, compiler_params=params)


_DOT = (((1,), (1,)), ((), ()))


def _tc_pre(x, W_r):
    N, D = x.shape
    BR = N

    def body(x_ref, wr_ref, r_ref):
        r_ref[...] = lax.dot_general(x_ref[...], wr_ref[...], _DOT,
                                     preferred_element_type=jnp.float32)

    return pl.pallas_call(
        body,
        grid=(N // BR,),
        in_specs=[pl.BlockSpec((BR, D), lambda i: (i, 0)),
                  pl.BlockSpec((D, D), lambda i: (0, 0))],
        out_specs=pl.BlockSpec((BR, D), lambda i: (i, 0)),
        out_shape=jax.ShapeDtypeStruct((N, D), jnp.float32),
    )(x, W_r)


def _tc_mid(s1, cntT, r1, W1_l, b1, W2_r):
    N, D = r1.shape
    BR = 2000

    def body(s_ref, c_ref, r_ref, wl_ref, b_ref, wr_ref, h_ref, rr_ref):
        s = s_ref[0] + s_ref[1]
        t = lax.dot_general(s, wl_ref[...], _DOT,
                            preferred_element_type=jnp.float32)
        c = jnp.sum(c_ref[...], axis=1, keepdims=True)
        pre = t / jnp.maximum(c, 1.0) + b_ref[...] + r_ref[...]
        nrm = jnp.sqrt(jnp.sum(pre * pre, axis=1, keepdims=True))
        h = jnp.maximum(pre / jnp.maximum(nrm, 1e-12), 0.0)
        h_ref[...] = h
        rr_ref[...] = lax.dot_general(h, wr_ref[...], _DOT,
                                      preferred_element_type=jnp.float32)

    return pl.pallas_call(
        body,
        grid=(N // BR,),
        in_specs=[pl.BlockSpec((NC, BR, D), lambda i: (0, i, 0)),
                  pl.BlockSpec((BR, NW), lambda i: (i, 0)),
                  pl.BlockSpec((BR, D), lambda i: (i, 0)),
                  pl.BlockSpec((D, D), lambda i: (0, 0)),
                  pl.BlockSpec((1, D), lambda i: (0, 0)),
                  pl.BlockSpec((D, D), lambda i: (0, 0))],
        out_specs=[pl.BlockSpec((BR, D), lambda i: (i, 0)),
                   pl.BlockSpec((BR, D), lambda i: (i, 0))],
        out_shape=[jax.ShapeDtypeStruct((N, D), jnp.float32)] * 2,
    )(s1, cntT, r1, W1_l, b1.reshape(1, D), W2_r)


def _tc_post(s2, cntT, r2, W2_l, b2):
    N, D = r2.shape
    BR = 2000

    def body(s_ref, c_ref, r_ref, wl_ref, b_ref, o_ref):
        s = s_ref[0] + s_ref[1]
        t = lax.dot_general(s, wl_ref[...], _DOT,
                            preferred_element_type=jnp.float32)
        c = jnp.sum(c_ref[...], axis=1, keepdims=True)
        o_ref[...] = t / jnp.maximum(c, 1.0) + b_ref[...] + r_ref[...]

    return pl.pallas_call(
        body,
        grid=(N // BR,),
        in_specs=[pl.BlockSpec((NC, BR, D), lambda i: (0, i, 0)),
                  pl.BlockSpec((BR, NW), lambda i: (i, 0)),
                  pl.BlockSpec((BR, D), lambda i: (i, 0)),
                  pl.BlockSpec((D, D), lambda i: (0, 0)),
                  pl.BlockSpec((1, D), lambda i: (0, 0))],
        out_specs=pl.BlockSpec((BR, D), lambda i: (i, 0)),
        out_shape=jax.ShapeDtypeStruct((N, D), jnp.float32),
    )(s2, cntT, r2, W2_l, b2.reshape(1, D))


def kernel(x, edge_index, W1_l, b1_l, W1_r, W2_l, b2_l, W2_r):
    N, D = x.shape
    E = edge_index.shape[1]
    src = edge_index[0].astype(jnp.int32)
    dst = edge_index[1].astype(jnp.int32)
    Npad = -(-N // 128) * 128
    zeros = jnp.zeros((Npad // NS, D), jnp.float32)

    part1, cnt = _make_agg(N, Npad, D, E, True)(x, src, dst, zeros)
    r1 = _tc_pre(x, W1_r)
    cntT = cnt.T
    h, r2 = _tc_mid(part1, cntT, r1, W1_l, b1_l, W2_r)
    part2, = _make_agg(N, Npad, D, E, False)(h, src, dst, zeros)
    return _tc_post(part2, cntT, r2, W2_l, b2_l)
